# baseline (speedup 1.0000x reference)
import jax
import jax.numpy as jnp
from jax import lax
from jax.experimental import pallas as pl
from jax.experimental.pallas import tpu as pltpu
from jax.experimental.pallas import tpu_sc as plsc

S, H, E, F = 2048, 768, 8, 3072
BT = 128
P = 3072
NBLK = P // BT
NC, NS = 2, 16
NW = NC * NS
RPW = P // NW
CH = RPW // 2
OUT_ROWS = S + 8


def _gelu(x):
    return 0.5 * x * (1.0 + jnp.tanh(0.7978845608028654 * x
                                     * (1.0 + 0.044715 * x * x)))


def _router_body(x_ref, rw_ref, rb_ref, b2_ref, prob_ref, ind_ref, outb_ref):
    logits = lax.dot_general(x_ref[...], rw_ref[...], (((1,), (1,)), ((), ())),
                             preferred_element_type=jnp.float32)
    logits = logits + rb_ref[...]
    m = jnp.max(logits, axis=1, keepdims=True)
    ex = jnp.exp(logits - m)
    route = ex / jnp.sum(ex, axis=1, keepdims=True)
    mp = jnp.max(route, axis=1, keepdims=True)
    prob_ref[...] = mp
    iota = lax.broadcasted_iota(jnp.int32, route.shape, 1)
    ind = jnp.min(jnp.where(route == mp, iota, E), axis=1, keepdims=True)
    ind_ref[...] = ind
    onehot = (iota == ind).astype(jnp.float32)
    outb_ref[...] = lax.dot_general(onehot, b2_ref[...],
                                    (((1,), (0,)), ((), ())),
                                    preferred_element_type=jnp.float32) * mp


def _router(x, rw, rb, b2):
    return pl.pallas_call(
        _router_body,
        out_shape=(
            jax.ShapeDtypeStruct((S, 1), jnp.float32),
            jax.ShapeDtypeStruct((S, 1), jnp.int32),
            jax.ShapeDtypeStruct((S, H), jnp.float32),
        ),
    )(x, rw, rb.reshape(1, E), b2)


def _dispatch_body(x_hbm, ind_hbm, prob_hbm,
                   xs_hbm, be_hbm, dest_hbm, probs_hbm,
                   ind_v, prob_v, seg_tok, seg_gat, seg_prob,
                   counts_l, misc_v, idx_v, rows_v,
                   counts_sh, gat_sh, sem):
    cid = lax.axis_index("c")
    sid = lax.axis_index("s")
    lane = lax.broadcasted_iota(jnp.int32, (16,), 0)

    @pl.when(sid < E)
    def _():
        pltpu.sync_copy(ind_hbm, ind_v)
        pltpu.sync_copy(prob_hbm, prob_v)

        def c_body(i, cntv):
            chunk = ind_v[pl.ds(i * 16, 16)]
            return cntv + plsc.all_reduce_population_count(chunk == sid)

        cntv = lax.fori_loop(0, S // 16, c_body, jnp.zeros((16,), jnp.int32))
        misc_v[pl.ds(0, 16)] = cntv
        pltpu.sync_copy(misc_v.at[pl.ds(0, 16)], counts_sh.at[sid])

    @pl.when(sid == E)
    def _():
        def z_body(i, _):
            seg_gat[pl.ds(i * 16, 16)] = jnp.zeros((16,), jnp.int32)
            return 0
        lax.fori_loop(0, S // 16, z_body, 0)
        pltpu.sync_copy(seg_gat.at[pl.ds(0, S)], gat_sh.at[pl.ds(0, S)])
        pltpu.sync_copy(seg_gat.at[pl.ds(0, P - S)], gat_sh.at[pl.ds(S, P - S)])

    plsc.subcore_barrier()

    pltpu.sync_copy(counts_sh, counts_l)
    cnts = jnp.zeros((16,), jnp.int32)
    for e in range(E):
        cnts = jnp.where(lane == e, counts_l[e], cnts)
    pc = ((cnts + (BT - 1)) >> 7) << 7
    poff_i = plsc.cumsum(pc)
    poff_x = poff_i - pc

    @pl.when(sid < E)
    def _():
        my_poff = pl.multiple_of(jnp.sum(jnp.where(lane == sid, poff_x, 0)),
                                 BT)
        my_pc = jnp.sum(jnp.where(lane == sid, pc, 0))

        def f_body(i, _):
            seg_tok[pl.ds(i * 16, 16)] = jnp.full((16,), S, jnp.int32)
            return 0
        lax.fori_loop(0, S // 16, f_body, 0)

        def r_body(i, base):
            chunk = ind_v[pl.ds(i * 16, 16)]
            m = chunk == sid
            ids = lane + i * 16
            pos = base + plsc.cumsum(m.astype(jnp.int32)) - 1
            plsc.store_scatter(seg_tok, [pos], ids, mask=m)
            return base + plsc.all_reduce_population_count(m)
        lax.fori_loop(0, S // 16, r_body, jnp.zeros((16,), jnp.int32))

        def g_body(i, _):
            tok = seg_tok[pl.ds(i * 16, 16)]
            valid = tok < S
            gidx = jnp.where(valid, tok, 0)
            seg_gat[pl.ds(i * 16, 16)] = gidx
            pv = plsc.load_gather(prob_v, [gidx])
            seg_prob[pl.ds(i * 16, 16)] = jnp.where(valid, pv, 0.0)
            return 0
        lax.fori_loop(0, my_pc >> 4, g_body, 0)

        def p_body(k, _):
            pltpu.sync_copy(
                seg_gat.at[pl.ds(k * BT, BT)],
                gat_sh.at[pl.ds(pl.multiple_of(my_poff + k * BT, BT), BT)])
            return 0
        lax.fori_loop(0, my_pc >> 7, p_body, 0)

        @pl.when(cid == 0)
        def _():
            def h_body(k, _):
                off = pl.multiple_of(my_poff + k * BT, BT)
                pltpu.sync_copy(seg_tok.at[pl.ds(k * BT, BT)],
                                dest_hbm.at[pl.ds(off, BT)])
                pltpu.sync_copy(seg_prob.at[pl.ds(k * BT, BT)],
                                probs_hbm.at[pl.ds(off, BT)])
                return 0
            lax.fori_loop(0, my_pc >> 7, h_body, 0)

    @pl.when(jnp.logical_and(sid == E + 1, cid == 0))
    def _():
        total = pl.multiple_of(jnp.sum(jnp.where(lane == E - 1, poff_i, 0)),
                               BT)
        s_sc = [jnp.sum(jnp.where(lane == e, poff_i, 0)) for e in range(E)]
        for c in range(2):
            i16 = lane + c * 16
            acc = jnp.zeros((16,), jnp.int32)
            for e in range(E):
                acc += (i16 * BT >= s_sc[e]).astype(jnp.int32)
            misc_v[pl.ds(c * 16, 16)] = jnp.minimum(acc, E - 1)
        pltpu.sync_copy(misc_v.at[pl.ds(0, 32)], be_hbm)

        def t0_body(i, _):
            seg_tok[pl.ds(i * 16, 16)] = jnp.full((16,), S, jnp.int32)
            seg_prob[pl.ds(i * 16, 16)] = jnp.zeros((16,), jnp.float32)
            return 0
        lax.fori_loop(0, BT // 16, t0_body, 0)

        def t_body(k, _):
            off = pl.multiple_of(total + k * BT, BT)
            pltpu.sync_copy(seg_tok.at[pl.ds(0, BT)],
                            dest_hbm.at[pl.ds(off, BT)])
            pltpu.sync_copy(seg_prob.at[pl.ds(0, BT)],
                            probs_hbm.at[pl.ds(off, BT)])
            return 0
        lax.fori_loop(0, (P - total) >> 7, t_body, 0)

    plsc.subcore_barrier()

    base = (cid * NS + sid) * RPW
    pltpu.sync_copy(gat_sh.at[pl.ds(base, RPW)], idx_v)
    pltpu.async_copy(x_hbm.at[idx_v], rows_v, sem).wait()
    pltpu.sync_copy(rows_v, xs_hbm.at[pl.ds(base, RPW)])


def _dispatch(x, ind, prob):
    mesh = plsc.VectorSubcoreMesh(core_axis_name="c", subcore_axis_name="s",
                                  num_cores=NC, num_subcores=NS)
    return pl.kernel(
        _dispatch_body,
        out_type=(
            jax.ShapeDtypeStruct((P, H), jnp.float32),
            jax.ShapeDtypeStruct((32,), jnp.int32),
            jax.ShapeDtypeStruct((P,), jnp.int32),
            jax.ShapeDtypeStruct((P,), jnp.float32),
        ),
        mesh=mesh,
        compiler_params=pltpu.CompilerParams(needs_layout_passes=False),
        scratch_types=[
            pltpu.VMEM((S,), jnp.int32),
            pltpu.VMEM((S,), jnp.float32),
            pltpu.VMEM((S,), jnp.int32),
            pltpu.VMEM((S,), jnp.int32),
            pltpu.VMEM((S,), jnp.float32),
            pltpu.VMEM((E, 16), jnp.int32),
            pltpu.VMEM((32,), jnp.int32),
            pltpu.VMEM((RPW,), jnp.int32),
            pltpu.VMEM((RPW, H), jnp.float32),
            pltpu.VMEM_SHARED((E, 16), jnp.int32),
            pltpu.VMEM_SHARED((P,), jnp.int32),
            pltpu.SemaphoreType.DMA,
        ],
    )(x, ind, prob)


def _mlp_body(be_ref, x_ref, w1_ref, b1_ref, w2_ref, p_ref, y_ref):
    h1 = lax.dot_general(x_ref[...], w1_ref[0], (((1,), (1,)), ((), ())),
                         preferred_element_type=jnp.float32)
    h1 = _gelu(h1 + b1_ref[0])
    y = lax.dot_general(h1, w2_ref[0], (((1,), (1,)), ((), ())),
                        preferred_element_type=jnp.float32)
    y_ref[...] = y * p_ref[...]


def _mlp_grouped(xs, w1, b1, w2, be, probs):
    gridspec = pltpu.PrefetchScalarGridSpec(
        num_scalar_prefetch=1,
        grid=(NBLK,),
        in_specs=[
            pl.BlockSpec((BT, H), lambda i, be: (i, 0)),
            pl.BlockSpec((1, F, H), lambda i, be: (be[i], 0, 0)),
            pl.BlockSpec((1, 1, F), lambda i, be: (be[i], 0, 0)),
            pl.BlockSpec((1, H, F), lambda i, be: (be[i], 0, 0)),
            pl.BlockSpec((BT, 1), lambda i, be: (i, 0)),
        ],
        out_specs=pl.BlockSpec((BT, H), lambda i, be: (i, 0)),
    )
    return pl.pallas_call(
        _mlp_body,
        grid_spec=gridspec,
        out_shape=jax.ShapeDtypeStruct((P, H), jnp.float32),
        compiler_params=pltpu.CompilerParams(
            dimension_semantics=("arbitrary",),
        ),
    )(be, xs, w1, b1.reshape(E, 1, F), w2, probs.reshape(P, 1))


def _scatter_body(y_hbm, dest_hbm, out_hbm, y_v, dest_v, destc_v, sem):
    cid = lax.axis_index("c")
    sid = lax.axis_index("s")
    base = (cid * NS + sid) * RPW
    pltpu.sync_copy(dest_hbm.at[pl.ds(base, RPW)], dest_v)
    for c in range(RPW // CH):
        pltpu.sync_copy(y_hbm.at[pl.ds(base + c * CH, CH)], y_v)

        def ix_body(q, _):
            destc_v[pl.ds(q * 16, 16)] = dest_v[pl.ds(c * CH + q * 16, 16)]
            return 0
        lax.fori_loop(0, CH // 16, ix_body, 0)
        pltpu.async_copy(y_v, out_hbm.at[destc_v], sem).wait()


def _scatter(y, dest):
    mesh = plsc.VectorSubcoreMesh(core_axis_name="c", subcore_axis_name="s",
                                  num_cores=NC, num_subcores=NS)
    return pl.kernel(
        _scatter_body,
        out_type=jax.ShapeDtypeStruct((OUT_ROWS, H), jnp.float32),
        mesh=mesh,
        compiler_params=pltpu.CompilerParams(needs_layout_passes=False),
        scratch_types=[
            pltpu.VMEM((CH, H), jnp.float32),
            pltpu.VMEM((RPW,), jnp.int32),
            pltpu.VMEM((CH,), jnp.int32),
            pltpu.SemaphoreType.DMA,
        ],
    )(y, dest)


def kernel(hidden_states, router_W, router_b, W1, b1, W2, b2):
    s, b, h = hidden_states.shape
    x = hidden_states.reshape(s * b, h)
    prob, ind, outb = _router(x, router_W, router_b, b2)
    xs, be, dest, probs = _dispatch(x, ind.reshape(S), prob.reshape(S))
    y = _mlp_grouped(xs, W1, b1, W2, be, probs)
    out = _scatter(y, dest)
    return out[:S].reshape(s, b, h), outb.reshape(s, b, h)


# speedup vs baseline: 1.0158x; 1.0158x over previous
import jax
import jax.numpy as jnp
from jax import lax
from jax.experimental import pallas as pl
from jax.experimental.pallas import tpu as pltpu
from jax.experimental.pallas import tpu_sc as plsc

S, H, E, F = 2048, 768, 8, 3072
BT = 128
P = 3072
NBLK = P // BT
NC, NS = 2, 16
NW = NC * NS
RPW = P // NW
CH = RPW // 2
OUT_ROWS = S + 8


def _gelu(x):
    return 0.5 * x * (1.0 + jnp.tanh(0.7978845608028654 * x
                                     * (1.0 + 0.044715 * x * x)))


def _router_body(x_ref, rw_ref, rb_ref, b2_ref, prob_ref, ind_ref, outb_ref):
    logits = lax.dot_general(x_ref[...], rw_ref[...], (((1,), (1,)), ((), ())),
                             preferred_element_type=jnp.float32)
    logits = logits + rb_ref[...]
    m = jnp.max(logits, axis=1, keepdims=True)
    ex = jnp.exp(logits - m)
    route = ex / jnp.sum(ex, axis=1, keepdims=True)
    mp = jnp.max(route, axis=1, keepdims=True)
    prob_ref[...] = mp
    iota = lax.broadcasted_iota(jnp.int32, route.shape, 1)
    ind = jnp.min(jnp.where(route == mp, iota, E), axis=1, keepdims=True)
    ind_ref[...] = ind
    onehot = (iota == ind).astype(jnp.float32)
    outb_ref[...] = lax.dot_general(onehot, b2_ref[...],
                                    (((1,), (0,)), ((), ())),
                                    preferred_element_type=jnp.float32) * mp


def _router(x, rw, rb, b2):
    return pl.pallas_call(
        _router_body,
        out_shape=(
            jax.ShapeDtypeStruct((S, 1), jnp.float32),
            jax.ShapeDtypeStruct((S, 1), jnp.int32),
            jax.ShapeDtypeStruct((S, H), jnp.float32),
        ),
    )(x, rw, rb.reshape(1, E), b2)


def _dispatch_body(x_hbm, ind_hbm, prob_hbm,
                   xs_hbm, be_hbm, dest_hbm, probs_hbm,
                   ind_v, prob_v, seg_tok,
                   counts_l, misc_v, tok_v, idx_v, dprob_v, rows_v,
                   counts_sh, tok_sh, sem, sem2):
    cid = lax.axis_index("c")
    sid = lax.axis_index("s")
    lane = lax.broadcasted_iota(jnp.int32, (16,), 0)

    ph = pltpu.async_copy(prob_hbm, prob_v, sem2)

    @pl.when(sid < E)
    def _():
        pltpu.sync_copy(ind_hbm, ind_v)

        def c_body(i, cntv):
            chunk = ind_v[pl.ds(i * 16, 16)]
            return cntv + plsc.all_reduce_population_count(chunk == sid)

        cntv = lax.fori_loop(0, S // 16, c_body, jnp.zeros((16,), jnp.int32))
        misc_v[pl.ds(0, 16)] = cntv
        pltpu.sync_copy(misc_v.at[pl.ds(0, 16)], counts_sh.at[sid])

    @pl.when(sid == E)
    def _():
        def z_body(i, _):
            seg_tok[pl.ds(i * 16, 16)] = jnp.full((16,), S, jnp.int32)
            return 0
        lax.fori_loop(0, S // 16, z_body, 0)
        pltpu.sync_copy(seg_tok.at[pl.ds(0, S)], tok_sh.at[pl.ds(0, S)])
        pltpu.sync_copy(seg_tok.at[pl.ds(0, P - S)], tok_sh.at[pl.ds(S, P - S)])

    plsc.subcore_barrier()

    pltpu.sync_copy(counts_sh, counts_l)
    cnts = jnp.zeros((16,), jnp.int32)
    for e in range(E):
        cnts = jnp.where(lane == e, counts_l[e], cnts)
    pc = ((cnts + (BT - 1)) >> 7) << 7
    poff_i = plsc.cumsum(pc)
    poff_x = poff_i - pc

    @pl.when(sid < E)
    def _():
        my_poff = pl.multiple_of(jnp.sum(jnp.where(lane == sid, poff_x, 0)),
                                 BT)
        my_pc = jnp.sum(jnp.where(lane == sid, pc, 0))

        def f_body(i, _):
            seg_tok[pl.ds(i * 16, 16)] = jnp.full((16,), S, jnp.int32)
            return 0
        lax.fori_loop(0, S // 16, f_body, 0)

        def r_body(i, base):
            chunk = ind_v[pl.ds(i * 16, 16)]
            m = chunk == sid
            ids = lane + i * 16
            pos = base + plsc.cumsum(m.astype(jnp.int32)) - 1
            plsc.store_scatter(seg_tok, [pos], ids, mask=m)
            return base + plsc.all_reduce_population_count(m)
        lax.fori_loop(0, S // 16, r_body, jnp.zeros((16,), jnp.int32))

        off = 0
        for sz in (2048, 1024, 512, 256, 128):
            @pl.when((my_pc & sz) != 0)
            def _(off=off, sz=sz):
                o = pl.multiple_of(off, BT)
                d = pl.multiple_of(my_poff + off, BT)
                pltpu.sync_copy(seg_tok.at[pl.ds(o, sz)],
                                tok_sh.at[pl.ds(d, sz)])
            off = off + (my_pc & sz)

    @pl.when(jnp.logical_and(sid == E + 1, cid == 0))
    def _():
        s_sc = [jnp.sum(jnp.where(lane == e, poff_i, 0)) for e in range(E)]
        for c in range(2):
            i16 = lane + c * 16
            acc = jnp.zeros((16,), jnp.int32)
            for e in range(E):
                acc += (i16 * BT >= s_sc[e]).astype(jnp.int32)
            misc_v[pl.ds(c * 16, 16)] = jnp.minimum(acc, E - 1)
        pltpu.sync_copy(misc_v.at[pl.ds(0, 32)], be_hbm)

    plsc.subcore_barrier()

    base = (cid * NS + sid) * RPW
    pltpu.sync_copy(tok_sh.at[pl.ds(base, RPW)], tok_v)
    ph.wait()
    for q in range(RPW // 16):
        tok = tok_v[pl.ds(q * 16, 16)]
        valid = tok < S
        gidx = jnp.where(valid, tok, 0)
        idx_v[pl.ds(q * 16, 16)] = gidx
        pv = plsc.load_gather(prob_v, [gidx])
        dprob_v[pl.ds(q * 16, 16)] = jnp.where(valid, pv, 0.0)
    wd = pltpu.async_copy(tok_v, dest_hbm.at[pl.ds(base, RPW)], sem2)
    wp = pltpu.async_copy(dprob_v, probs_hbm.at[pl.ds(base, RPW)], sem2)
    pltpu.async_copy(x_hbm.at[idx_v], rows_v, sem).wait()
    pltpu.sync_copy(rows_v, xs_hbm.at[pl.ds(base, RPW)])
    wd.wait()
    wp.wait()


def _dispatch(x, ind, prob):
    mesh = plsc.VectorSubcoreMesh(core_axis_name="c", subcore_axis_name="s",
                                  num_cores=NC, num_subcores=NS)
    return pl.kernel(
        _dispatch_body,
        out_type=(
            jax.ShapeDtypeStruct((P, H), jnp.float32),
            jax.ShapeDtypeStruct((32,), jnp.int32),
            jax.ShapeDtypeStruct((P,), jnp.int32),
            jax.ShapeDtypeStruct((P,), jnp.float32),
        ),
        mesh=mesh,
        compiler_params=pltpu.CompilerParams(needs_layout_passes=False),
        scratch_types=[
            pltpu.VMEM((S,), jnp.int32),
            pltpu.VMEM((S,), jnp.float32),
            pltpu.VMEM((S,), jnp.int32),
            pltpu.VMEM((E, 16), jnp.int32),
            pltpu.VMEM((32,), jnp.int32),
            pltpu.VMEM((RPW,), jnp.int32),
            pltpu.VMEM((RPW,), jnp.int32),
            pltpu.VMEM((RPW,), jnp.float32),
            pltpu.VMEM((RPW, H), jnp.float32),
            pltpu.VMEM_SHARED((E, 16), jnp.int32),
            pltpu.VMEM_SHARED((P,), jnp.int32),
            pltpu.SemaphoreType.DMA,
            pltpu.SemaphoreType.DMA,
        ],
    )(x, ind, prob)


def _mlp_body(be_ref, x_ref, w1_ref, b1_ref, w2_ref, p_ref, y_ref):
    h1 = lax.dot_general(x_ref[...], w1_ref[0], (((1,), (1,)), ((), ())),
                         preferred_element_type=jnp.float32)
    h1 = _gelu(h1 + b1_ref[0])
    y = lax.dot_general(h1, w2_ref[0], (((1,), (1,)), ((), ())),
                        preferred_element_type=jnp.float32)
    y_ref[...] = y * p_ref[...]


def _mlp_grouped(xs, w1, b1, w2, be, probs):
    gridspec = pltpu.PrefetchScalarGridSpec(
        num_scalar_prefetch=1,
        grid=(NBLK,),
        in_specs=[
            pl.BlockSpec((BT, H), lambda i, be: (i, 0)),
            pl.BlockSpec((1, F, H), lambda i, be: (be[i], 0, 0)),
            pl.BlockSpec((1, 1, F), lambda i, be: (be[i], 0, 0)),
            pl.BlockSpec((1, H, F), lambda i, be: (be[i], 0, 0)),
            pl.BlockSpec((BT, 1), lambda i, be: (i, 0)),
        ],
        out_specs=pl.BlockSpec((BT, H), lambda i, be: (i, 0)),
    )
    return pl.pallas_call(
        _mlp_body,
        grid_spec=gridspec,
        out_shape=jax.ShapeDtypeStruct((P, H), jnp.float32),
        compiler_params=pltpu.CompilerParams(
            dimension_semantics=("arbitrary",),
        ),
    )(be, xs, w1, b1.reshape(E, 1, F), w2, probs.reshape(P, 1))


def _scatter_body(y_hbm, dest_hbm, out_hbm,
                  y0_v, y1_v, dest_v, destc0, destc1, sem, sem2):
    cid = lax.axis_index("c")
    sid = lax.axis_index("s")
    base = (cid * NS + sid) * RPW
    pltpu.sync_copy(dest_hbm.at[pl.ds(base, RPW)], dest_v)
    h0 = pltpu.async_copy(y_hbm.at[pl.ds(base, CH)], y0_v, sem)
    h1 = pltpu.async_copy(y_hbm.at[pl.ds(base + CH, CH)], y1_v, sem2)
    for q in range(CH // 16):
        destc0[pl.ds(q * 16, 16)] = dest_v[pl.ds(q * 16, 16)]
        destc1[pl.ds(q * 16, 16)] = dest_v[pl.ds(CH + q * 16, 16)]
    h0.wait()
    s0 = pltpu.async_copy(y0_v, out_hbm.at[destc0], sem)
    h1.wait()
    s1 = pltpu.async_copy(y1_v, out_hbm.at[destc1], sem2)
    s0.wait()
    s1.wait()


def _scatter(y, dest):
    mesh = plsc.VectorSubcoreMesh(core_axis_name="c", subcore_axis_name="s",
                                  num_cores=NC, num_subcores=NS)
    return pl.kernel(
        _scatter_body,
        out_type=jax.ShapeDtypeStruct((OUT_ROWS, H), jnp.float32),
        mesh=mesh,
        compiler_params=pltpu.CompilerParams(needs_layout_passes=False),
        scratch_types=[
            pltpu.VMEM((CH, H), jnp.float32),
            pltpu.VMEM((CH, H), jnp.float32),
            pltpu.VMEM((RPW,), jnp.int32),
            pltpu.VMEM((CH,), jnp.int32),
            pltpu.VMEM((CH,), jnp.int32),
            pltpu.SemaphoreType.DMA,
            pltpu.SemaphoreType.DMA,
        ],
    )(y, dest)


def kernel(hidden_states, router_W, router_b, W1, b1, W2, b2):
    s, b, h = hidden_states.shape
    x = hidden_states.reshape(s * b, h)
    prob, ind, outb = _router(x, router_W, router_b, b2)
    xs, be, dest, probs = _dispatch(x, ind.reshape(S), prob.reshape(S))
    y = _mlp_grouped(xs, W1, b1, W2, be, probs)
    out = _scatter(y, dest)
    return out[:S].reshape(s, b, h), outb.reshape(s, b, h)


# speedup vs baseline: 1.3891x; 1.3676x over previous
import jax
import jax.numpy as jnp
from jax import lax
from jax.experimental import pallas as pl
from jax.experimental.pallas import tpu as pltpu
from jax.experimental.pallas import tpu_sc as plsc

S, H, E, F = 2048, 768, 8, 3072
BT = 128
P = 3072
NBLK = P // BT
NC, NS = 2, 16
RPW = P // (NC * NS)


def _gelu(x):
    return 0.5 * x * (1.0 + jnp.tanh(0.7978845608028654 * x
                                     * (1.0 + 0.044715 * x * x)))


def _router_body(x_ref, rw_ref, rb_ref, b2_ref, prob_ref, ind_ref, outb_ref):
    logits = lax.dot_general(x_ref[...], rw_ref[...], (((1,), (1,)), ((), ())),
                             preferred_element_type=jnp.float32)
    logits = logits + rb_ref[...]
    m = jnp.max(logits, axis=1, keepdims=True)
    ex = jnp.exp(logits - m)
    route = ex / jnp.sum(ex, axis=1, keepdims=True)
    mp = jnp.max(route, axis=1, keepdims=True)
    prob_ref[...] = mp
    iota = lax.broadcasted_iota(jnp.int32, route.shape, 1)
    ind = jnp.min(jnp.where(route == mp, iota, E), axis=1, keepdims=True)
    ind_ref[...] = ind
    onehot = (iota == ind).astype(jnp.float32)
    outb_ref[...] = lax.dot_general(onehot, b2_ref[...],
                                    (((1,), (0,)), ((), ())),
                                    preferred_element_type=jnp.float32) * mp


def _router(x, rw, rb, b2):
    return pl.pallas_call(
        _router_body,
        out_shape=(
            jax.ShapeDtypeStruct((S, 1), jnp.float32),
            jax.ShapeDtypeStruct((S, 1), jnp.int32),
            jax.ShapeDtypeStruct((S, H), jnp.float32),
        ),
    )(x, rw, rb.reshape(1, E), b2)


def _dispatch_body(ind_hbm,
                   be_hbm, dest_hbm, gat_hbm,
                   ind_v, seg_tok, counts_l, misc_v, tok_v, idx_v,
                   counts_sh, tok_sh, sem, sem2):
    cid = lax.axis_index("c")
    sid = lax.axis_index("s")
    lane = lax.broadcasted_iota(jnp.int32, (16,), 0)

    @pl.when(sid < E)
    def _():
        pltpu.sync_copy(ind_hbm, ind_v)

        def c_body(i, cntv):
            chunk = ind_v[pl.ds(i * 16, 16)]
            return cntv + plsc.all_reduce_population_count(chunk == sid)

        cntv = lax.fori_loop(0, S // 16, c_body, jnp.zeros((16,), jnp.int32))
        misc_v[pl.ds(0, 16)] = cntv
        pltpu.sync_copy(misc_v.at[pl.ds(0, 16)], counts_sh.at[sid])

    @pl.when(sid == E)
    def _():
        def z_body(i, _):
            seg_tok[pl.ds(i * 16, 16)] = jnp.full((16,), S, jnp.int32)
            return 0
        lax.fori_loop(0, S // 16, z_body, 0)
        pltpu.sync_copy(seg_tok.at[pl.ds(0, S)], tok_sh.at[pl.ds(0, S)])
        pltpu.sync_copy(seg_tok.at[pl.ds(0, P - S)], tok_sh.at[pl.ds(S, P - S)])

    plsc.subcore_barrier()

    pltpu.sync_copy(counts_sh, counts_l)
    cnts = jnp.zeros((16,), jnp.int32)
    for e in range(E):
        cnts = jnp.where(lane == e, counts_l[e], cnts)
    pc = ((cnts + (BT - 1)) >> 7) << 7
    poff_i = plsc.cumsum(pc)
    poff_x = poff_i - pc

    @pl.when(sid < E)
    def _():
        my_poff = pl.multiple_of(jnp.sum(jnp.where(lane == sid, poff_x, 0)),
                                 BT)
        my_pc = jnp.sum(jnp.where(lane == sid, pc, 0))

        def f_body(i, _):
            seg_tok[pl.ds(i * 16, 16)] = jnp.full((16,), S, jnp.int32)
            return 0
        lax.fori_loop(0, S // 16, f_body, 0)

        def r_body(i, base):
            chunk = ind_v[pl.ds(i * 16, 16)]
            m = chunk == sid
            ids = lane + i * 16
            pos = base + plsc.cumsum(m.astype(jnp.int32)) - 1
            plsc.store_scatter(seg_tok, [pos], ids, mask=m)
            return base + plsc.all_reduce_population_count(m)
        lax.fori_loop(0, S // 16, r_body, jnp.zeros((16,), jnp.int32))

        off = 0
        for sz in (2048, 1024, 512, 256, 128):
            @pl.when((my_pc & sz) != 0)
            def _(off=off, sz=sz):
                o = pl.multiple_of(off, BT)
                d = pl.multiple_of(my_poff + off, BT)
                pltpu.sync_copy(seg_tok.at[pl.ds(o, sz)],
                                tok_sh.at[pl.ds(d, sz)])
            off = off + (my_pc & sz)

    @pl.when(jnp.logical_and(sid == E + 1, cid == 0))
    def _():
        s_sc = [jnp.sum(jnp.where(lane == e, poff_i, 0)) for e in range(E)]
        for c in range(2):
            i16 = lane + c * 16
            acc = jnp.zeros((16,), jnp.int32)
            for e in range(E):
                acc += (i16 * BT >= s_sc[e]).astype(jnp.int32)
            misc_v[pl.ds(c * 16, 16)] = jnp.minimum(acc, E - 1)
        pltpu.sync_copy(misc_v.at[pl.ds(0, 32)], be_hbm)

    plsc.subcore_barrier()

    base = (cid * NS + sid) * RPW
    pltpu.sync_copy(tok_sh.at[pl.ds(base, RPW)], tok_v)
    for q in range(RPW // 16):
        tok = tok_v[pl.ds(q * 16, 16)]
        idx_v[pl.ds(q * 16, 16)] = jnp.where(tok < S, tok, 0)
    wd = pltpu.async_copy(tok_v, dest_hbm.at[pl.ds(base, RPW)], sem)
    wg = pltpu.async_copy(idx_v, gat_hbm.at[pl.ds(base, RPW)], sem2)
    wd.wait()
    wg.wait()


def _dispatch(ind):
    mesh = plsc.VectorSubcoreMesh(core_axis_name="c", subcore_axis_name="s",
                                  num_cores=NC, num_subcores=NS)
    return pl.kernel(
        _dispatch_body,
        out_type=(
            jax.ShapeDtypeStruct((32,), jnp.int32),
            jax.ShapeDtypeStruct((P,), jnp.int32),
            jax.ShapeDtypeStruct((P,), jnp.int32),
        ),
        mesh=mesh,
        compiler_params=pltpu.CompilerParams(needs_layout_passes=False),
        scratch_types=[
            pltpu.VMEM((S,), jnp.int32),
            pltpu.VMEM((S,), jnp.int32),
            pltpu.VMEM((E, 16), jnp.int32),
            pltpu.VMEM((32,), jnp.int32),
            pltpu.VMEM((RPW,), jnp.int32),
            pltpu.VMEM((RPW,), jnp.int32),
            pltpu.VMEM_SHARED((E, 16), jnp.int32),
            pltpu.VMEM_SHARED((P,), jnp.int32),
            pltpu.SemaphoreType.DMA,
            pltpu.SemaphoreType.DMA,
        ],
    )(ind)


def _mlp_body(be_ref, gat_ref, x_ref, w1_ref, b1_ref, w2_ref, y_ref):
    iota = lax.broadcasted_iota(jnp.int32, (BT, S), 1)
    onehot = (iota == gat_ref[...]).astype(jnp.float32)
    xs = lax.dot_general(onehot, x_ref[...], (((1,), (0,)), ((), ())),
                         preferred_element_type=jnp.float32)
    h1 = lax.dot_general(xs, w1_ref[0], (((1,), (1,)), ((), ())),
                         preferred_element_type=jnp.float32)
    h1 = _gelu(h1 + b1_ref[0])
    y_ref[...] = lax.dot_general(h1, w2_ref[0], (((1,), (1,)), ((), ())),
                                 preferred_element_type=jnp.float32)


def _mlp_grouped(x, w1, b1, w2, be, gat):
    gridspec = pltpu.PrefetchScalarGridSpec(
        num_scalar_prefetch=1,
        grid=(NBLK,),
        in_specs=[
            pl.BlockSpec((BT, 1), lambda i, be: (i, 0)),
            pl.BlockSpec((S, H), lambda i, be: (0, 0)),
            pl.BlockSpec((1, F, H), lambda i, be: (be[i], 0, 0)),
            pl.BlockSpec((1, 1, F), lambda i, be: (be[i], 0, 0)),
            pl.BlockSpec((1, H, F), lambda i, be: (be[i], 0, 0)),
        ],
        out_specs=pl.BlockSpec((BT, H), lambda i, be: (i, 0)),
    )
    return pl.pallas_call(
        _mlp_body,
        grid_spec=gridspec,
        out_shape=jax.ShapeDtypeStruct((P, H), jnp.float32),
        compiler_params=pltpu.CompilerParams(
            dimension_semantics=("arbitrary",),
        ),
    )(be, gat.reshape(P, 1), x, w1, b1.reshape(E, 1, F), w2)


def _combine_body(dest_ref, y_ref, prob_ref, out_ref):
    t = pl.program_id(0)
    iota = lax.broadcasted_iota(jnp.int32, (BT, P), 0) + t * BT
    onehot = (iota == dest_ref[...].reshape(1, P)).astype(jnp.float32)
    out = lax.dot_general(onehot, y_ref[...], (((1,), (0,)), ((), ())),
                          preferred_element_type=jnp.float32)
    out_ref[...] = out * prob_ref[...]


def _combine(y, dest, prob):
    return pl.pallas_call(
        _combine_body,
        grid=(S // BT,),
        in_specs=[
            pl.BlockSpec((P, 1), lambda t: (0, 0)),
            pl.BlockSpec((P, H), lambda t: (0, 0)),
            pl.BlockSpec((BT, 1), lambda t: (t, 0)),
        ],
        out_specs=pl.BlockSpec((BT, H), lambda t: (t, 0)),
        out_shape=jax.ShapeDtypeStruct((S, H), jnp.float32),
        compiler_params=pltpu.CompilerParams(
            dimension_semantics=("arbitrary",),
        ),
    )(dest.reshape(P, 1), y, prob)


def kernel(hidden_states, router_W, router_b, W1, b1, W2, b2):
    s, b, h = hidden_states.shape
    x = hidden_states.reshape(s * b, h)
    prob, ind, outb = _router(x, router_W, router_b, b2)
    be, dest, gat = _dispatch(ind.reshape(S))
    y = _mlp_grouped(x, W1, b1, W2, be, gat)
    out = _combine(y, dest, prob)
    return out.reshape(s, b, h), outb.reshape(s, b, h)


# speedup vs baseline: 1.4753x; 1.0620x over previous
import jax
import jax.numpy as jnp
from jax import lax
from jax.experimental import pallas as pl
from jax.experimental.pallas import tpu as pltpu
from jax.experimental.pallas import tpu_sc as plsc

S, H, E, F = 2048, 768, 8, 3072
BT = 128
P = 3072
NBLK = P // BT
NC, NS = 2, 16
RPW = P // (NC * NS)


def _gelu(x):
    return 0.5 * x * (1.0 + jnp.tanh(0.7978845608028654 * x
                                     * (1.0 + 0.044715 * x * x)))


def _router_body(x_ref, rw_ref, rb_ref, b2_ref, prob_ref, ind_ref, outb_ref):
    x = x_ref[...].reshape(S, H)
    logits = lax.dot_general(x, rw_ref[...], (((1,), (1,)), ((), ())),
                             preferred_element_type=jnp.float32)
    logits = logits + rb_ref[...]
    m = jnp.max(logits, axis=1, keepdims=True)
    ex = jnp.exp(logits - m)
    route = ex / jnp.sum(ex, axis=1, keepdims=True)
    mp = jnp.max(route, axis=1, keepdims=True)
    prob_ref[...] = mp
    iota = lax.broadcasted_iota(jnp.int32, route.shape, 1)
    ind = jnp.min(jnp.where(route == mp, iota, E), axis=1, keepdims=True)
    ind_ref[...] = ind
    onehot = (iota == ind).astype(jnp.float32)
    outb = lax.dot_general(onehot, b2_ref[...], (((1,), (0,)), ((), ())),
                           preferred_element_type=jnp.float32) * mp
    outb_ref[...] = outb.reshape(S, 1, H)


def _router(x, rw, rb, b2):
    return pl.pallas_call(
        _router_body,
        out_shape=(
            jax.ShapeDtypeStruct((S, 1), jnp.float32),
            jax.ShapeDtypeStruct((S, 1), jnp.int32),
            jax.ShapeDtypeStruct((S, 1, H), jnp.float32),
        ),
    )(x, rw, rb.reshape(1, E), b2)


def _dispatch_body(ind_hbm,
                   be_hbm, dest_hbm, gat_hbm,
                   ind_v, seg_tok, counts_l, misc_v, tok_v, idx_v,
                   counts_sh, tok_sh, sem, sem2):
    cid = lax.axis_index("c")
    sid = lax.axis_index("s")
    lane = lax.broadcasted_iota(jnp.int32, (16,), 0)

    @pl.when(sid < E)
    def _():
        pltpu.sync_copy(ind_hbm, ind_v)

        def c_body(i, cntv):
            chunk = ind_v[pl.ds(i * 16, 16)]
            return cntv + plsc.all_reduce_population_count(chunk == sid)

        cntv = lax.fori_loop(0, S // 16, c_body, jnp.zeros((16,), jnp.int32))
        misc_v[pl.ds(0, 16)] = cntv
        pltpu.sync_copy(misc_v.at[pl.ds(0, 16)], counts_sh.at[sid])

    @pl.when(sid == E)
    def _():
        def z_body(i, _):
            seg_tok[pl.ds(i * 16, 16)] = jnp.full((16,), S, jnp.int32)
            return 0
        lax.fori_loop(0, S // 16, z_body, 0)
        pltpu.sync_copy(seg_tok.at[pl.ds(0, S)], tok_sh.at[pl.ds(0, S)])
        pltpu.sync_copy(seg_tok.at[pl.ds(0, P - S)], tok_sh.at[pl.ds(S, P - S)])

    plsc.subcore_barrier()

    pltpu.sync_copy(counts_sh, counts_l)
    cnts = jnp.zeros((16,), jnp.int32)
    for e in range(E):
        cnts = jnp.where(lane == e, counts_l[e], cnts)
    pc = ((cnts + (BT - 1)) >> 7) << 7
    poff_i = plsc.cumsum(pc)
    poff_x = poff_i - pc

    @pl.when(sid < E)
    def _():
        my_poff = pl.multiple_of(jnp.sum(jnp.where(lane == sid, poff_x, 0)),
                                 BT)
        my_pc = jnp.sum(jnp.where(lane == sid, pc, 0))

        def f_body(i, _):
            seg_tok[pl.ds(i * 16, 16)] = jnp.full((16,), S, jnp.int32)
            return 0
        lax.fori_loop(0, S // 16, f_body, 0)

        def r_body(i, base):
            chunk = ind_v[pl.ds(i * 16, 16)]
            m = chunk == sid
            ids = lane + i * 16
            pos = base + plsc.cumsum(m.astype(jnp.int32)) - 1
            plsc.store_scatter(seg_tok, [pos], ids, mask=m)
            return base + plsc.all_reduce_population_count(m)
        lax.fori_loop(0, S // 16, r_body, jnp.zeros((16,), jnp.int32))

        off = 0
        for sz in (2048, 1024, 512, 256, 128):
            @pl.when((my_pc & sz) != 0)
            def _(off=off, sz=sz):
                o = pl.multiple_of(off, BT)
                d = pl.multiple_of(my_poff + off, BT)
                pltpu.sync_copy(seg_tok.at[pl.ds(o, sz)],
                                tok_sh.at[pl.ds(d, sz)])
            off = off + (my_pc & sz)

    @pl.when(jnp.logical_and(sid == E + 1, cid == 0))
    def _():
        s_sc = [jnp.sum(jnp.where(lane == e, poff_i, 0)) for e in range(E)]
        for c in range(2):
            i16 = lane + c * 16
            acc = jnp.zeros((16,), jnp.int32)
            for e in range(E):
                acc += (i16 * BT >= s_sc[e]).astype(jnp.int32)
            misc_v[pl.ds(c * 16, 16)] = jnp.minimum(acc, E - 1)
        pltpu.sync_copy(misc_v.at[pl.ds(0, 32)], be_hbm)

    plsc.subcore_barrier()

    base = (cid * NS + sid) * RPW
    pltpu.sync_copy(tok_sh.at[pl.ds(base, RPW)], tok_v)
    for q in range(RPW // 16):
        tok = tok_v[pl.ds(q * 16, 16)]
        idx_v[pl.ds(q * 16, 16)] = jnp.where(tok < S, tok, 0)
    wd = pltpu.async_copy(tok_v, dest_hbm.at[pl.ds(base, RPW)], sem)
    wg = pltpu.async_copy(idx_v, gat_hbm.at[pl.ds(base, RPW)], sem2)
    wd.wait()
    wg.wait()


def _dispatch(ind):
    mesh = plsc.VectorSubcoreMesh(core_axis_name="c", subcore_axis_name="s",
                                  num_cores=NC, num_subcores=NS)
    return pl.kernel(
        _dispatch_body,
        out_type=(
            jax.ShapeDtypeStruct((32,), jnp.int32),
            jax.ShapeDtypeStruct((P,), jnp.int32),
            jax.ShapeDtypeStruct((P,), jnp.int32),
        ),
        mesh=mesh,
        compiler_params=pltpu.CompilerParams(needs_layout_passes=False),
        scratch_types=[
            pltpu.VMEM((S,), jnp.int32),
            pltpu.VMEM((S,), jnp.int32),
            pltpu.VMEM((E, 16), jnp.int32),
            pltpu.VMEM((32,), jnp.int32),
            pltpu.VMEM((RPW,), jnp.int32),
            pltpu.VMEM((RPW,), jnp.int32),
            pltpu.VMEM_SHARED((E, 16), jnp.int32),
            pltpu.VMEM_SHARED((P,), jnp.int32),
            pltpu.SemaphoreType.DMA,
            pltpu.SemaphoreType.DMA,
        ],
    )(ind)


def _mlp_body(be_ref, gat_ref, x_ref, w1_ref, b1_ref, w2_ref, y_ref):
    iota = lax.broadcasted_iota(jnp.int32, (BT, S), 1)
    onehot = (iota == gat_ref[...]).astype(jnp.float32)
    xs = lax.dot_general(onehot, x_ref[...].reshape(S, H),
                         (((1,), (0,)), ((), ())),
                         preferred_element_type=jnp.float32)
    h1 = lax.dot_general(xs, w1_ref[0], (((1,), (1,)), ((), ())),
                         preferred_element_type=jnp.float32)
    h1 = _gelu(h1 + b1_ref[0])
    y = lax.dot_general(h1, w2_ref[0], (((1,), (1,)), ((), ())),
                         preferred_element_type=jnp.float32)
    y_ref[...] = y.astype(jnp.bfloat16)


def _mlp_grouped(x, w1, b1, w2, be, gat):
    gridspec = pltpu.PrefetchScalarGridSpec(
        num_scalar_prefetch=1,
        grid=(NBLK,),
        in_specs=[
            pl.BlockSpec((BT, 1), lambda i, be: (i, 0)),
            pl.BlockSpec((S, 1, H), lambda i, be: (0, 0, 0)),
            pl.BlockSpec((1, F, H), lambda i, be: (be[i], 0, 0)),
            pl.BlockSpec((1, 1, F), lambda i, be: (be[i], 0, 0)),
            pl.BlockSpec((1, H, F), lambda i, be: (be[i], 0, 0)),
        ],
        out_specs=pl.BlockSpec((BT, H), lambda i, be: (i, 0)),
    )
    return pl.pallas_call(
        _mlp_body,
        grid_spec=gridspec,
        out_shape=jax.ShapeDtypeStruct((P, H), jnp.bfloat16),
        compiler_params=pltpu.CompilerParams(
            dimension_semantics=("arbitrary",),
        ),
    )(be, gat.reshape(P, 1), x, w1, b1.reshape(E, 1, F), w2)


def _combine_body(dest_ref, y_ref, prob_ref, out_ref):
    t = pl.program_id(0)
    iota = lax.broadcasted_iota(jnp.int32, (BT, P), 0) + t * BT
    onehot = (iota == dest_ref[...].reshape(1, P)).astype(jnp.bfloat16)
    out = lax.dot_general(onehot, y_ref[...], (((1,), (0,)), ((), ())),
                          preferred_element_type=jnp.float32)
    out_ref[...] = (out * prob_ref[...]).reshape(BT, 1, H)


def _combine(y, dest, prob):
    return pl.pallas_call(
        _combine_body,
        grid=(S // BT,),
        in_specs=[
            pl.BlockSpec((P, 1), lambda t: (0, 0)),
            pl.BlockSpec((P, H), lambda t: (0, 0)),
            pl.BlockSpec((BT, 1), lambda t: (t, 0)),
        ],
        out_specs=pl.BlockSpec((BT, 1, H), lambda t: (t, 0, 0)),
        out_shape=jax.ShapeDtypeStruct((S, 1, H), jnp.float32),
        compiler_params=pltpu.CompilerParams(
            dimension_semantics=("arbitrary",),
        ),
    )(dest.reshape(P, 1), y, prob)


def kernel(hidden_states, router_W, router_b, W1, b1, W2, b2):
    s, b, h = hidden_states.shape
    prob, ind, outb = _router(hidden_states, router_W, router_b, b2)
    be, dest, gat = _dispatch(ind.reshape(S))
    y = _mlp_grouped(hidden_states, W1, b1, W2, be, gat)
    out = _combine(y, dest, prob)
    return out.reshape(s, b, h), outb.reshape(s, b, h)
